# baseline (speedup 1.0000x reference)
import functools

import jax
import jax.numpy as jnp
from jax import lax
from jax.experimental import pallas as pl
from jax.experimental.pallas import tpu as pltpu
from jax.experimental.pallas import tpu_sc as plsc

N_NODES = 10000
N_EDGES = 160000
D = 256
D2 = 128
D2P = D // 2

NC, NS, L = 2, 16, 16
NW = NC * NS

NSL = 5
ESL = N_EDGES // NSL

_mesh = plsc.VectorSubcoreMesh(core_axis_name="c", subcore_axis_name="s")

GCH = 40

_GATHER_SCRATCH = [
    pltpu.VMEM((GCH,), jnp.int32),
    pltpu.VMEM((GCH,), jnp.int32),
    pltpu.VMEM((GCH,), jnp.int32),
    pltpu.VMEM((GCH,), jnp.int32),
    pltpu.VMEM((GCH, D2P), jnp.uint32),
    pltpu.VMEM((GCH, D2P), jnp.uint32),
    pltpu.VMEM((GCH, D2P), jnp.uint32),
    pltpu.VMEM((GCH, D2P), jnp.uint32),
    pltpu.SemaphoreType.DMA,
    pltpu.SemaphoreType.DMA,
    pltpu.SemaphoreType.DMA,
    pltpu.SemaphoreType.DMA,
]


def _make_gather(base_edge, esl):
    epw = esl // NW
    gnch = epw // GCH
    assert gnch % 2 == 1

    def body(sproj, rproj, senders, receivers, gs, gr,
             sidx0, sidx1, ridx0, ridx1, sbuf0, sbuf1, rbuf0, rbuf1,
             gsem0, gsem1, wsem0, wsem1):
        sidx, ridx = (sidx0, sidx1), (ridx0, ridx1)
        sbuf, rbuf = (sbuf0, sbuf1), (rbuf0, rbuf1)
        gsem, wsem = (gsem0, gsem1), (wsem0, wsem1)
        wid = lax.axis_index("s") * NC + lax.axis_index("c")
        base = wid * epw

        def load_idx(k, b):
            off = base_edge + base + k * GCH
            pltpu.sync_copy(senders.at[pl.ds(off, GCH)], sidx[b])
            pltpu.sync_copy(receivers.at[pl.ds(off, GCH)], ridx[b])

        def start_gather(b):
            pltpu.async_copy(sproj.at[sidx[b]], sbuf[b], gsem[b])
            pltpu.async_copy(rproj.at[ridx[b]], rbuf[b], gsem[b])

        def wait_gather(b):
            pltpu.make_async_copy(sproj.at[sidx[b]], sbuf[b], gsem[b]).wait()
            pltpu.make_async_copy(rproj.at[ridx[b]], rbuf[b], gsem[b]).wait()

        def start_wb(k, b):
            off = base + k * GCH
            pltpu.async_copy(sbuf[b], gs.at[pl.ds(off, GCH)], wsem[b])
            pltpu.async_copy(rbuf[b], gr.at[pl.ds(off, GCH)], wsem[b])

        def wait_wb(b):
            pltpu.make_async_copy(sbuf[b], gs.at[pl.ds(0, GCH)], wsem[b]).wait()
            pltpu.make_async_copy(rbuf[b], gr.at[pl.ds(0, GCH)], wsem[b]).wait()

        load_idx(0, 0)
        start_gather(0)

        @pl.loop(0, gnch - 1, step=2)
        def _(k):
            load_idx(k + 1, 1)

            @pl.when(k > 0)
            def _():
                wait_wb(1)
            start_gather(1)
            wait_gather(0)
            start_wb(k, 0)
            load_idx(k + 2, 0)
            wait_wb(0)
            start_gather(0)
            wait_gather(1)
            start_wb(k + 1, 1)

        wait_gather(0)
        start_wb(gnch - 1, 0)
        wait_wb(0)
        wait_wb(1)

    return pl.kernel(
        body, mesh=_mesh,
        out_type=[jax.ShapeDtypeStruct((esl, D2P), jnp.uint32),
                  jax.ShapeDtypeStruct((esl, D2P), jnp.uint32)],
        scratch_types=_GATHER_SCRATCH,
    )


EPT = ESL // NS
SCH = 80
SNCH = EPT // SCH
assert SNCH % 2 == 1
ZR = 80
NZCH = N_NODES // ZR


NB = 4


def _make_scatter(bases):
    nrefs = len(bases)

    def body(*args):
        t_refs = args[:nrefs]
        receivers, out, acc = args[nrefs:nrefs + 3]
        ridx = args[nrefs + 3:nrefs + 3 + NB]
        tbuf = args[nrefs + 3 + NB:nrefs + 3 + 2 * NB]
        tsem = args[nrefs + 3 + 2 * NB:nrefs + 3 + 3 * NB]
        ssem = args[nrefs + 3 + 3 * NB:nrefs + 3 + 4 * NB]
        zbuf = tbuf[0]
        c = lax.axis_index("c")
        s = lax.axis_index("s")

        @pl.loop(0, ZR)
        def _(i):
            @pl.loop(0, D2 // L)
            def _(j):
                zbuf[i, pl.ds(j * L, L)] = jnp.zeros((L,), jnp.float32)

        @pl.loop(s, NZCH, step=NS)
        def _(z):
            pltpu.sync_copy(zbuf, acc.at[pl.ds(z * ZR, ZR)])

        plsc.subcore_barrier()

        for t, base_edge in zip(t_refs, bases):
            tb = s * EPT

            def load_t(k, b):
                off = tb + k * SCH
                pltpu.sync_copy(receivers.at[pl.ds(base_edge + off, SCH)], ridx[b])
                pltpu.async_copy(t.at[pl.ds(off, SCH), pl.ds(c * D2, D2)],
                                 tbuf[b], tsem[b])

            def wait_t(b):
                pltpu.make_async_copy(t.at[pl.ds(0, SCH), pl.ds(0, D2)],
                                      tbuf[b], tsem[b]).wait()

            def start_scat(b):
                pltpu.async_copy(tbuf[b], acc.at[ridx[b]], ssem[b], add=True)

            def wait_scat(b):
                pltpu.make_async_copy(tbuf[b], acc.at[ridx[b]], ssem[b]).wait()

            load_t(0, 0)
            load_t(1, 1)
            assert (SNCH - 1) % NB == 0

            @pl.loop(0, SNCH - 1, step=NB)
            def _(k):
                for b in range(NB):
                    @pl.when(k + b >= 2)
                    def _():
                        wait_scat((b + 2) % NB)
                    @pl.when(k + b + 2 < SNCH)
                    def _():
                        load_t(k + b + 2, (b + 2) % NB)
                    wait_t(b)
                    start_scat(b)

            j = SNCH - 1
            b = j % NB
            wait_scat((b + 2) % NB)
            wait_t(b)
            start_scat(b)
            wait_scat((j - 1) % NB)
            wait_scat(b)

        plsc.subcore_barrier()

        @pl.loop(s, NZCH, step=NS)
        def _(z):
            pltpu.sync_copy(acc.at[pl.ds(z * ZR, ZR)], out.at[c, pl.ds(z * ZR, ZR)])

    scratch = (
        [pltpu.VMEM_SHARED((N_NODES, D2), jnp.float32)]
        + [pltpu.VMEM((SCH,), jnp.int32) for _ in range(NB)]
        + [pltpu.VMEM((SCH, D2), jnp.float32) for _ in range(NB)]
        + [pltpu.SemaphoreType.DMA for _ in range(2 * NB)]
    )
    return pl.kernel(
        body, mesh=_mesh,
        out_type=jax.ShapeDtypeStruct((NC, N_NODES, D2), jnp.float32),
        scratch_types=scratch,
    )


PBLK = 1000
EBLK = 1000


def _round_bf16_bits(x):
    u = lax.bitcast_convert_type(x, jnp.uint32)
    return (u + 0x7FFF + ((u >> 16) & 1)) >> 16


def _proj_tc(se, re, ws, bs, wr, sp, rp):
    s = jnp.dot(se[...], ws[...], preferred_element_type=jnp.float32) + bs[...]
    r = jnp.dot(re[...], wr[...], preferred_element_type=jnp.float32)
    sp[...] = _round_bf16_bits(s[:, :D2]) | (_round_bf16_bits(s[:, D2:]) << 16)
    rp[...] = _round_bf16_bits(r[:, :D2]) | (_round_bf16_bits(r[:, D2:]) << 16)


def _unpack(p):
    even = lax.bitcast_convert_type(p << 16, jnp.float32)
    odd = lax.bitcast_convert_type((p >> 16) << 16, jnp.float32)
    return even, odd


def _edge_tc(gs, gr, ee, we, lns, lnb, t):
    gse, gso = _unpack(gs[...])
    gre, gro = _unpack(gr[...])
    xe = gse + gre
    xo = gso + gro
    mean = (jnp.sum(xe, axis=-1, keepdims=True)
            + jnp.sum(xo, axis=-1, keepdims=True)) * (1.0 / D)
    xe = xe - mean
    xo = xo - mean
    var = (jnp.sum(xe * xe, axis=-1, keepdims=True)
           + jnp.sum(xo * xo, axis=-1, keepdims=True)) * (1.0 / D)
    x = jnp.concatenate([xe, xo], axis=1)
    x = x * lax.rsqrt(var + 1e-6) * lns[...] + lnb[...]
    x = x * jax.nn.sigmoid(x)
    ep = jnp.dot(ee[...].astype(jnp.bfloat16), we[...].astype(jnp.bfloat16),
                 preferred_element_type=jnp.float32)
    t[...] = x * ep


def _out_tc(p0, p1, nrm, wout, o):
    m = p0[...] + p1[...]
    x = jnp.concatenate([m[0], m[1]], axis=1) * nrm[...]
    y = jnp.dot(x, wout[...], preferred_element_type=jnp.float32)
    o[...] = y * jax.nn.sigmoid(y)


def kernel(s_embed, r_embed, e_embed, norm, senders, receivers,
           W_s, b_s, W_r, ln_scale, ln_bias, W_e, W_out):
    f32 = jnp.float32
    senders = senders.astype(jnp.int32)
    receivers = receivers.astype(jnp.int32)

    def colperm(w):
        return jnp.concatenate([w[:, ::2], w[:, 1::2]], axis=1)

    W_s_p = colperm(W_s)
    W_r_p = colperm(W_r)
    W_e_p = colperm(W_e)
    b_s_p = jnp.concatenate([b_s[::2], b_s[1::2]]).reshape(1, D)
    lns_p = jnp.concatenate([ln_scale[::2], ln_scale[1::2]]).reshape(1, D)
    lnb_p = jnp.concatenate([ln_bias[::2], ln_bias[1::2]]).reshape(1, D)
    W_out_p = jnp.concatenate([W_out[::2, :], W_out[1::2, :]], axis=0)

    full = pl.BlockSpec((D, D), lambda i: (0, 0))
    row_vec = pl.BlockSpec((1, D), lambda i: (0, 0))

    sproj, rproj = pl.pallas_call(
        _proj_tc,
        grid=(N_NODES // PBLK,),
        in_specs=[
            pl.BlockSpec((PBLK, D), lambda i: (i, 0)),
            pl.BlockSpec((PBLK, D), lambda i: (i, 0)),
            full, row_vec, full,
        ],
        out_specs=[pl.BlockSpec((PBLK, D2P), lambda i: (i, 0)),
                   pl.BlockSpec((PBLK, D2P), lambda i: (i, 0))],
        out_shape=[jax.ShapeDtypeStruct((N_NODES, D2P), jnp.uint32),
                   jax.ShapeDtypeStruct((N_NODES, D2P), jnp.uint32)],
    )(s_embed, r_embed, W_s_p, b_s_p, W_r_p)

    gather_slices = [(0, ESL), (ESL, ESL), (2 * ESL, 3 * ESL)]
    garrs = []
    for base, esl in gather_slices:
        garrs.append(_make_gather(base, esl)(sproj, rproj, senders, receivers))

    gmap = [(0, 0), (1, 0), (2, 0), (2, ESL // EBLK), (2, 2 * (ESL // EBLK))]
    ts = []
    for i in range(NSL):
        gi, boff = gmap[i]
        gs, gr = garrs[gi]
        blk0 = i * (ESL // EBLK)
        t = pl.pallas_call(
            _edge_tc,
            grid=(ESL // EBLK,),
            in_specs=[
                pl.BlockSpec((EBLK, D2P), lambda j, b=boff: (j + b, 0)),
                pl.BlockSpec((EBLK, D2P), lambda j, b=boff: (j + b, 0)),
                pl.BlockSpec((EBLK, D), lambda j, b=blk0: (j + b, 0)),
                full, row_vec, row_vec,
            ],
            out_specs=pl.BlockSpec((EBLK, D), lambda j: (j, 0)),
            out_shape=jax.ShapeDtypeStruct((ESL, D), f32),
        )(gs, gr, e_embed, W_e_p, lns_p, lnb_p)
        ts.append(t)

    p0 = _make_scatter([0, ESL, 2 * ESL, 3 * ESL])(ts[0], ts[1], ts[2], ts[3],
                                                   receivers)
    p1 = _make_scatter([4 * ESL])(ts[4], receivers)

    out = pl.pallas_call(
        _out_tc,
        grid=(N_NODES // PBLK,),
        in_specs=[
            pl.BlockSpec((NC, PBLK, D2), lambda i: (0, i, 0)),
            pl.BlockSpec((NC, PBLK, D2), lambda i: (0, i, 0)),
            pl.BlockSpec((PBLK, 1), lambda i: (i, 0)),
            full,
        ],
        out_specs=pl.BlockSpec((PBLK, D), lambda i: (i, 0)),
        out_shape=jax.ShapeDtypeStruct((N_NODES, D), f32),
    )(p0, p1, norm.reshape(N_NODES, 1), W_out_p)

    return out


# speedup vs baseline: 1.0716x; 1.0716x over previous
import functools

import jax
import jax.numpy as jnp
from jax import lax
from jax.experimental import pallas as pl
from jax.experimental.pallas import tpu as pltpu
from jax.experimental.pallas import tpu_sc as plsc

N_NODES = 10000
N_EDGES = 160000
D = 256
D2 = 128
D2P = D // 2

NC, NS, L = 2, 16, 16
NW = NC * NS

NSL = 5
ESL = N_EDGES // NSL

_mesh = plsc.VectorSubcoreMesh(core_axis_name="c", subcore_axis_name="s")

GCH = 40

_GATHER_SCRATCH = [
    pltpu.VMEM((GCH,), jnp.int32),
    pltpu.VMEM((GCH,), jnp.int32),
    pltpu.VMEM((GCH,), jnp.int32),
    pltpu.VMEM((GCH,), jnp.int32),
    pltpu.VMEM((GCH, D2P), jnp.uint32),
    pltpu.VMEM((GCH, D2P), jnp.uint32),
    pltpu.VMEM((GCH, D2P), jnp.uint32),
    pltpu.VMEM((GCH, D2P), jnp.uint32),
    pltpu.SemaphoreType.DMA,
    pltpu.SemaphoreType.DMA,
    pltpu.SemaphoreType.DMA,
    pltpu.SemaphoreType.DMA,
]


def _make_gather(base_edge, esl):
    epw = esl // NW
    gnch = epw // GCH
    assert gnch % 2 == 1

    def body(sproj, rproj, senders, receivers, gs, gr,
             sidx0, sidx1, ridx0, ridx1, sbuf0, sbuf1, rbuf0, rbuf1,
             gsem0, gsem1, wsem0, wsem1):
        sidx, ridx = (sidx0, sidx1), (ridx0, ridx1)
        sbuf, rbuf = (sbuf0, sbuf1), (rbuf0, rbuf1)
        gsem, wsem = (gsem0, gsem1), (wsem0, wsem1)
        wid = lax.axis_index("s") * NC + lax.axis_index("c")
        base = wid * epw

        def load_idx(k, b):
            off = base_edge + base + k * GCH
            pltpu.sync_copy(senders.at[pl.ds(off, GCH)], sidx[b])
            pltpu.sync_copy(receivers.at[pl.ds(off, GCH)], ridx[b])

        def start_gather(b):
            pltpu.async_copy(sproj.at[sidx[b]], sbuf[b], gsem[b])
            pltpu.async_copy(rproj.at[ridx[b]], rbuf[b], gsem[b])

        def wait_gather(b):
            pltpu.make_async_copy(sproj.at[sidx[b]], sbuf[b], gsem[b]).wait()
            pltpu.make_async_copy(rproj.at[ridx[b]], rbuf[b], gsem[b]).wait()

        def start_wb(k, b):
            off = base + k * GCH
            pltpu.async_copy(sbuf[b], gs.at[pl.ds(off, GCH)], wsem[b])
            pltpu.async_copy(rbuf[b], gr.at[pl.ds(off, GCH)], wsem[b])

        def wait_wb(b):
            pltpu.make_async_copy(sbuf[b], gs.at[pl.ds(0, GCH)], wsem[b]).wait()
            pltpu.make_async_copy(rbuf[b], gr.at[pl.ds(0, GCH)], wsem[b]).wait()

        load_idx(0, 0)
        start_gather(0)

        @pl.loop(0, gnch - 1, step=2)
        def _(k):
            load_idx(k + 1, 1)

            @pl.when(k > 0)
            def _():
                wait_wb(1)
            start_gather(1)
            wait_gather(0)
            start_wb(k, 0)
            load_idx(k + 2, 0)
            wait_wb(0)
            start_gather(0)
            wait_gather(1)
            start_wb(k + 1, 1)

        wait_gather(0)
        start_wb(gnch - 1, 0)
        wait_wb(0)
        wait_wb(1)

    return pl.kernel(
        body, mesh=_mesh,
        out_type=[jax.ShapeDtypeStruct((esl, D2P), jnp.uint32),
                  jax.ShapeDtypeStruct((esl, D2P), jnp.uint32)],
        scratch_types=_GATHER_SCRATCH,
    )


EPT = ESL // NS
SCH = 80
SNCH = EPT // SCH
assert SNCH % 2 == 1
ZR = 80
NZCH = N_NODES // ZR


NB = 4


def _make_scatter(bases):
    nrefs = len(bases)

    def body(*args):
        t_refs = args[:nrefs]
        receivers, out, acc = args[nrefs:nrefs + 3]
        ridx = args[nrefs + 3:nrefs + 3 + NB]
        tbuf = args[nrefs + 3 + NB:nrefs + 3 + 2 * NB]
        tsem = args[nrefs + 3 + 2 * NB:nrefs + 3 + 3 * NB]
        ssem = args[nrefs + 3 + 3 * NB:nrefs + 3 + 4 * NB]
        zbuf = tbuf[0]
        c = lax.axis_index("c")
        s = lax.axis_index("s")

        @pl.loop(0, ZR)
        def _(i):
            @pl.loop(0, D2 // L)
            def _(j):
                zbuf[i, pl.ds(j * L, L)] = jnp.zeros((L,), jnp.float32)

        @pl.loop(s, NZCH, step=NS)
        def _(z):
            pltpu.sync_copy(zbuf, acc.at[pl.ds(z * ZR, ZR)])

        plsc.subcore_barrier()

        for t, base_edge in zip(t_refs, bases):
            tb = s * EPT

            def load_t(k, b):
                off = tb + k * SCH
                pltpu.sync_copy(receivers.at[pl.ds(base_edge + off, SCH)], ridx[b])
                pltpu.async_copy(t.at[pl.ds(off, SCH), pl.ds(c * D2, D2)],
                                 tbuf[b], tsem[b])

            def wait_t(b):
                pltpu.make_async_copy(t.at[pl.ds(0, SCH), pl.ds(0, D2)],
                                      tbuf[b], tsem[b]).wait()

            def start_scat(b):
                pltpu.async_copy(tbuf[b], acc.at[ridx[b]], ssem[b], add=True)

            def wait_scat(b):
                pltpu.make_async_copy(tbuf[b], acc.at[ridx[b]], ssem[b]).wait()

            load_t(0, 0)
            load_t(1, 1)
            assert (SNCH - 1) % NB == 0

            @pl.loop(0, SNCH - 1, step=NB)
            def _(k):
                for b in range(NB):
                    @pl.when(k + b >= 2)
                    def _():
                        wait_scat((b + 2) % NB)
                    @pl.when(k + b + 2 < SNCH)
                    def _():
                        load_t(k + b + 2, (b + 2) % NB)
                    wait_t(b)
                    start_scat(b)

            j = SNCH - 1
            b = j % NB
            wait_scat((b + 2) % NB)
            wait_t(b)
            start_scat(b)
            wait_scat((j - 1) % NB)
            wait_scat(b)

        plsc.subcore_barrier()

        @pl.loop(s, NZCH, step=NS)
        def _(z):
            pltpu.sync_copy(acc.at[pl.ds(z * ZR, ZR)], out.at[c, pl.ds(z * ZR, ZR)])

    scratch = (
        [pltpu.VMEM_SHARED((N_NODES, D2), jnp.float32)]
        + [pltpu.VMEM((SCH,), jnp.int32) for _ in range(NB)]
        + [pltpu.VMEM((SCH, D2), jnp.float32) for _ in range(NB)]
        + [pltpu.SemaphoreType.DMA for _ in range(2 * NB)]
    )
    return pl.kernel(
        body, mesh=_mesh,
        out_type=jax.ShapeDtypeStruct((NC, N_NODES, D2), jnp.float32),
        scratch_types=scratch,
    )


PBLK = 1000
EBLK = 1000


def _round_bf16_bits(x):
    u = lax.bitcast_convert_type(x, jnp.uint32)
    return (u + 0x7FFF + ((u >> 16) & 1)) >> 16


def _proj_tc(se, re, ws, bs, wr, sp, rp):
    s = jnp.dot(se[...], ws[...], preferred_element_type=jnp.float32) + bs[...]
    r = jnp.dot(re[...], wr[...], preferred_element_type=jnp.float32)
    sp[...] = _round_bf16_bits(s[:, :D2]) | (_round_bf16_bits(s[:, D2:]) << 16)
    rp[...] = _round_bf16_bits(r[:, :D2]) | (_round_bf16_bits(r[:, D2:]) << 16)


def _unpack(p):
    even = lax.bitcast_convert_type(p << 16, jnp.float32)
    odd = lax.bitcast_convert_type((p >> 16) << 16, jnp.float32)
    return even, odd


def _edge_tc(gs, gr, ee, we, lns, lnb, t):
    gse, gso = _unpack(gs[...])
    gre, gro = _unpack(gr[...])
    xe = gse + gre
    xo = gso + gro
    mean = (jnp.sum(xe, axis=-1, keepdims=True)
            + jnp.sum(xo, axis=-1, keepdims=True)) * (1.0 / D)
    xe = xe - mean
    xo = xo - mean
    var = (jnp.sum(xe * xe, axis=-1, keepdims=True)
           + jnp.sum(xo * xo, axis=-1, keepdims=True)) * (1.0 / D)
    x = jnp.concatenate([xe, xo], axis=1)
    x = x * lax.rsqrt(var + 1e-6) * lns[...] + lnb[...]
    x = x * jax.nn.sigmoid(x)
    ep = jnp.dot(ee[...].astype(jnp.bfloat16), we[...].astype(jnp.bfloat16),
                 preferred_element_type=jnp.float32)
    t[...] = x * ep


def _out_tc(p0, p1, p2, nrm, wout, o):
    m = p0[...] + p1[...] + p2[...]
    x = jnp.concatenate([m[0], m[1]], axis=1) * nrm[...]
    y = jnp.dot(x, wout[...], preferred_element_type=jnp.float32)
    o[...] = y * jax.nn.sigmoid(y)


def kernel(s_embed, r_embed, e_embed, norm, senders, receivers,
           W_s, b_s, W_r, ln_scale, ln_bias, W_e, W_out):
    f32 = jnp.float32
    senders = senders.astype(jnp.int32)
    receivers = receivers.astype(jnp.int32)

    def colperm(w):
        return jnp.concatenate([w[:, ::2], w[:, 1::2]], axis=1)

    W_s_p = colperm(W_s)
    W_r_p = colperm(W_r)
    W_e_p = colperm(W_e)
    b_s_p = jnp.concatenate([b_s[::2], b_s[1::2]]).reshape(1, D)
    lns_p = jnp.concatenate([ln_scale[::2], ln_scale[1::2]]).reshape(1, D)
    lnb_p = jnp.concatenate([ln_bias[::2], ln_bias[1::2]]).reshape(1, D)
    W_out_p = jnp.concatenate([W_out[::2, :], W_out[1::2, :]], axis=0)

    full = pl.BlockSpec((D, D), lambda i: (0, 0))
    row_vec = pl.BlockSpec((1, D), lambda i: (0, 0))

    sproj, rproj = pl.pallas_call(
        _proj_tc,
        grid=(N_NODES // PBLK,),
        in_specs=[
            pl.BlockSpec((PBLK, D), lambda i: (i, 0)),
            pl.BlockSpec((PBLK, D), lambda i: (i, 0)),
            full, row_vec, full,
        ],
        out_specs=[pl.BlockSpec((PBLK, D2P), lambda i: (i, 0)),
                   pl.BlockSpec((PBLK, D2P), lambda i: (i, 0))],
        out_shape=[jax.ShapeDtypeStruct((N_NODES, D2P), jnp.uint32),
                   jax.ShapeDtypeStruct((N_NODES, D2P), jnp.uint32)],
    )(s_embed, r_embed, W_s_p, b_s_p, W_r_p)

    gather_slices = [(0, ESL), (ESL, ESL), (2 * ESL, 3 * ESL)]
    garrs = []
    for base, esl in gather_slices:
        garrs.append(_make_gather(base, esl)(sproj, rproj, senders, receivers))

    gmap = [(0, 0), (1, 0), (2, 0), (2, ESL // EBLK), (2, 2 * (ESL // EBLK))]
    ts = []
    for i in range(NSL):
        gi, boff = gmap[i]
        gs, gr = garrs[gi]
        blk0 = i * (ESL // EBLK)
        t = pl.pallas_call(
            _edge_tc,
            grid=(ESL // EBLK,),
            in_specs=[
                pl.BlockSpec((EBLK, D2P), lambda j, b=boff: (j + b, 0)),
                pl.BlockSpec((EBLK, D2P), lambda j, b=boff: (j + b, 0)),
                pl.BlockSpec((EBLK, D), lambda j, b=blk0: (j + b, 0)),
                full, row_vec, row_vec,
            ],
            out_specs=pl.BlockSpec((EBLK, D), lambda j: (j, 0)),
            out_shape=jax.ShapeDtypeStruct((ESL, D), f32),
        )(gs, gr, e_embed, W_e_p, lns_p, lnb_p)
        ts.append(t)

    p0 = _make_scatter([0, ESL])(ts[0], ts[1], receivers)
    p1 = _make_scatter([2 * ESL, 3 * ESL])(ts[2], ts[3], receivers)
    p2 = _make_scatter([4 * ESL])(ts[4], receivers)

    out = pl.pallas_call(
        _out_tc,
        grid=(N_NODES // PBLK,),
        in_specs=[
            pl.BlockSpec((NC, PBLK, D2), lambda i: (0, i, 0)),
            pl.BlockSpec((NC, PBLK, D2), lambda i: (0, i, 0)),
            pl.BlockSpec((NC, PBLK, D2), lambda i: (0, i, 0)),
            pl.BlockSpec((PBLK, 1), lambda i: (i, 0)),
            full,
        ],
        out_specs=pl.BlockSpec((PBLK, D), lambda i: (i, 0)),
        out_shape=jax.ShapeDtypeStruct((N_NODES, D), f32),
    )(p0, p1, p2, norm.reshape(N_NODES, 1), W_out_p)

    return out
